# baseline (device time: 302918 ns/iter reference)
import jax
import jax.numpy as jnp
from jax import lax
from jax.experimental import pallas as pl
from jax.experimental.pallas import tpu as pltpu

N_DEV = 32
B, SQ, D_MODEL = 2, 512, 768
H_LOC, DH = 8, 64
WINDOW = 128
CHUNK = SQ // N_DEV


def kernel(x, Wq, K_ext, V_ext, Wo):
    my = lax.axis_index("i")
    k_loc = lax.dynamic_slice_in_dim(K_ext, my * H_LOC, H_LOC, axis=2)
    v_loc = lax.dynamic_slice_in_dim(V_ext, my * H_LOC, H_LOC, axis=2)

    def body(x_ref, wq_ref, k_ref, v_ref, wo_ref, out_ref,
             ctx_ref, sbuf_ref, rs_recv_ref,
             rs_send_sems, rs_recv_sems, ag_send_sems, ag_recv_sems):
        my_pos = lax.axis_index("i")
        left = lax.rem(my_pos + N_DEV - 1, N_DEV)
        right = lax.rem(my_pos + 1, N_DEV)

        barrier_sem = pltpu.get_barrier_semaphore()
        for nbr in (left, right):
            pl.semaphore_signal(
                barrier_sem, inc=1,
                device_id=(nbr,), device_id_type=pl.DeviceIdType.MESH,
            )
        pl.semaphore_wait(barrier_sem, 2)

        x2 = x_ref[...].reshape(B * SQ, D_MODEL)
        q = jnp.dot(x2, wq_ref[...], preferred_element_type=jnp.float32)

        qi = lax.broadcasted_iota(jnp.int32, (SQ, SQ), 0)
        ki = lax.broadcasted_iota(jnp.int32, (SQ, SQ), 1)
        mask = jnp.abs(qi - ki) <= WINDOW

        for b in range(B):
            for h in range(H_LOC):
                qh = q[b * SQ:(b + 1) * SQ, h * DH:(h + 1) * DH]
                kh = k_ref[b, :, h, :]
                s = lax.dot_general(
                    qh, kh, (((1,), (1,)), ((), ())),
                    preferred_element_type=jnp.float32,
                ) * 0.125
                s = jnp.where(mask, s, -1e9)
                m = jnp.max(s, axis=1, keepdims=True)
                w = jnp.exp(s - m)
                w = w / jnp.sum(w, axis=1, keepdims=True)
                ctx_ref[b, :, h * DH:(h + 1) * DH] = jnp.dot(
                    w, v_ref[b, :, h, :], preferred_element_type=jnp.float32
                )
        for b in range(B):
            out_ref[b, :, :] = jnp.dot(
                ctx_ref[b, :, :], wo_ref[...],
                preferred_element_type=jnp.float32,
            )

        for s in range(N_DEV - 1):
            c_send = lax.rem(my_pos - s + N_DEV, N_DEV)
            local_chunk = out_ref[:, pl.ds(c_send * CHUNK, CHUNK), :]
            if s == 0:
                sbuf_ref[0] = local_chunk
            else:
                sbuf_ref[s] = rs_recv_ref[s - 1] + local_chunk
            rdma = pltpu.make_async_remote_copy(
                src_ref=sbuf_ref.at[s],
                dst_ref=rs_recv_ref.at[s],
                send_sem=rs_send_sems.at[s],
                recv_sem=rs_recv_sems.at[s],
                device_id=(right,),
                device_id_type=pl.DeviceIdType.MESH,
            )
            rdma.start()
            rdma.wait()

        c_own = lax.rem(my_pos + 1, N_DEV)
        out_ref[:, pl.ds(c_own * CHUNK, CHUNK), :] = (
            rs_recv_ref[N_DEV - 2]
            + out_ref[:, pl.ds(c_own * CHUNK, CHUNK), :]
        )

        for t in range(N_DEV - 1):
            c = lax.rem(my_pos + 1 - t + N_DEV, N_DEV)
            rdma = pltpu.make_async_remote_copy(
                src_ref=out_ref.at[:, pl.ds(c * CHUNK, CHUNK), :],
                dst_ref=out_ref.at[:, pl.ds(c * CHUNK, CHUNK), :],
                send_sem=ag_send_sems.at[t],
                recv_sem=ag_recv_sems.at[t],
                device_id=(right,),
                device_id_type=pl.DeviceIdType.MESH,
            )
            rdma.start()
            rdma.wait()

    return pl.pallas_call(
        body,
        out_shape=jax.ShapeDtypeStruct((B, SQ, D_MODEL), jnp.float32),
        in_specs=[pl.BlockSpec(memory_space=pltpu.VMEM)] * 5,
        out_specs=pl.BlockSpec(memory_space=pltpu.VMEM),
        scratch_shapes=[
            pltpu.VMEM((B, SQ, H_LOC * DH), jnp.float32),
            pltpu.VMEM((N_DEV - 1, B, CHUNK, D_MODEL), jnp.float32),
            pltpu.VMEM((N_DEV - 1, B, CHUNK, D_MODEL), jnp.float32),
            pltpu.SemaphoreType.DMA((N_DEV - 1,)),
            pltpu.SemaphoreType.DMA((N_DEV - 1,)),
            pltpu.SemaphoreType.DMA((N_DEV - 1,)),
            pltpu.SemaphoreType.DMA((N_DEV - 1,)),
        ],
        compiler_params=pltpu.CompilerParams(collective_id=0),
    )(x, Wq, k_loc, v_loc, Wo)


# device time: 215099 ns/iter; 1.4083x vs baseline; 1.4083x over previous
import jax
import jax.numpy as jnp
from jax import lax
from jax.experimental import pallas as pl
from jax.experimental.pallas import tpu as pltpu

N_DEV = 32
B, SQ, D_MODEL = 2, 512, 768
H_LOC, DH = 8, 64
WINDOW = 128
ZROWS, YROWS, XROWS = 128, 32, 16


def kernel(x, Wq, K_ext, V_ext, Wo):
    my = lax.axis_index("i")
    k_loc = lax.dynamic_slice_in_dim(K_ext, my * H_LOC, H_LOC, axis=2)
    v_loc = lax.dynamic_slice_in_dim(V_ext, my * H_LOC, H_LOC, axis=2)

    def body(x_ref, wq_ref, k_ref, v_ref, wo_ref, out_ref,
             ctx_ref, sbz, rcz, sby, rcy, sbx, rcx,
             send_sems, recv_sems):
        p = lax.axis_index("i")
        mz = p // 8
        r = lax.rem(p, 8)
        my_y = r // 2
        mx = lax.rem(lax.rem(r, 2) + lax.rem(my_y, 2), 2)

        def pos(xx, yy, zz):
            return 8 * zz + 2 * yy + lax.rem(xx + lax.rem(yy, 2), 2)

        z_next = pos(mx, my_y, lax.rem(mz + 1, 4))
        z_prev = pos(mx, my_y, lax.rem(mz + 3, 4))
        y_next = pos(mx, lax.rem(my_y + 1, 4), mz)
        y_prev = pos(mx, lax.rem(my_y + 3, 4), mz)
        x_part = pos(1 - mx, my_y, mz)

        partners = (z_next, z_prev, y_next, y_prev, x_part)
        barrier_sem = pltpu.get_barrier_semaphore()
        for nbr in partners:
            pl.semaphore_signal(
                barrier_sem, inc=1,
                device_id=(nbr,), device_id_type=pl.DeviceIdType.MESH,
            )
        pl.semaphore_wait(barrier_sem, len(partners))

        x2 = x_ref[...].reshape(B * SQ, D_MODEL)
        q = jnp.dot(x2, wq_ref[...], preferred_element_type=jnp.float32)

        qi = lax.broadcasted_iota(jnp.int32, (SQ, SQ), 0)
        ki = lax.broadcasted_iota(jnp.int32, (SQ, SQ), 1)
        mask = jnp.abs(qi - ki) <= WINDOW

        for b in range(B):
            for h in range(H_LOC):
                qh = q[b * SQ:(b + 1) * SQ, h * DH:(h + 1) * DH]
                kh = k_ref[b, :, h, :]
                s = lax.dot_general(
                    qh, kh, (((1,), (1,)), ((), ())),
                    preferred_element_type=jnp.float32,
                ) * 0.125
                s = jnp.where(mask, s, -1e9)
                m = jnp.max(s, axis=1, keepdims=True)
                w = jnp.exp(s - m)
                w = w / jnp.sum(w, axis=1, keepdims=True)
                ctx_ref[b, :, h * DH:(h + 1) * DH] = jnp.dot(
                    w, v_ref[b, :, h, :], preferred_element_type=jnp.float32
                )
        for b in range(B):
            out_ref[b, :, :] = jnp.dot(
                ctx_ref[b, :, :], wo_ref[...],
                preferred_element_type=jnp.float32,
            )

        def rdma_step(src, dst, sem_idx, target):
            op = pltpu.make_async_remote_copy(
                src_ref=src, dst_ref=dst,
                send_sem=send_sems.at[sem_idx],
                recv_sem=recv_sems.at[sem_idx],
                device_id=(target,),
                device_id_type=pl.DeviceIdType.MESH,
            )
            op.start()
            op.wait()

        for s in range(3):
            g = lax.rem(mz - s + 4, 4)
            loc = out_ref[:, pl.ds(g * ZROWS, ZROWS), :]
            sbz[s] = loc if s == 0 else rcz[s - 1] + loc
            rdma_step(sbz.at[s], rcz.at[s], s, z_next)
        own_z = lax.rem(mz + 1, 4)
        zrow = own_z * ZROWS
        out_ref[:, pl.ds(zrow, ZROWS), :] = (
            out_ref[:, pl.ds(zrow, ZROWS), :] + rcz[2]
        )

        for s in range(3):
            gy = lax.rem(my_y - s + 4, 4)
            loc = out_ref[:, pl.ds(zrow + gy * YROWS, YROWS), :]
            sby[s] = loc if s == 0 else rcy[s - 1] + loc
            rdma_step(sby.at[s], rcy.at[s], 3 + s, y_next)
        own_y = lax.rem(my_y + 1, 4)
        yrow = zrow + own_y * YROWS
        out_ref[:, pl.ds(yrow, YROWS), :] = (
            out_ref[:, pl.ds(yrow, YROWS), :] + rcy[2]
        )

        gx = 1 - mx
        sbx[0] = out_ref[:, pl.ds(yrow + gx * XROWS, XROWS), :]
        rdma_step(sbx.at[0], rcx.at[0], 6, x_part)
        own_row = yrow + mx * XROWS
        out_ref[:, pl.ds(own_row, XROWS), :] = (
            out_ref[:, pl.ds(own_row, XROWS), :] + rcx[0]
        )

        rdma_step(
            out_ref.at[:, pl.ds(own_row, XROWS), :],
            out_ref.at[:, pl.ds(own_row, XROWS), :],
            7, x_part,
        )
        for t in range(3):
            gy = lax.rem(my_y + 1 - t + 4, 4)
            row0 = zrow + gy * YROWS
            rdma_step(
                out_ref.at[:, pl.ds(row0, YROWS), :],
                out_ref.at[:, pl.ds(row0, YROWS), :],
                8 + t, y_next,
            )
        for t in range(3):
            g = lax.rem(mz + 1 - t + 4, 4)
            row0 = g * ZROWS
            rdma_step(
                out_ref.at[:, pl.ds(row0, ZROWS), :],
                out_ref.at[:, pl.ds(row0, ZROWS), :],
                11 + t, z_next,
            )

    return pl.pallas_call(
        body,
        out_shape=jax.ShapeDtypeStruct((B, SQ, D_MODEL), jnp.float32),
        in_specs=[pl.BlockSpec(memory_space=pltpu.VMEM)] * 5,
        out_specs=pl.BlockSpec(memory_space=pltpu.VMEM),
        scratch_shapes=[
            pltpu.VMEM((B, SQ, H_LOC * DH), jnp.float32),
            pltpu.VMEM((3, B, ZROWS, D_MODEL), jnp.float32),
            pltpu.VMEM((3, B, ZROWS, D_MODEL), jnp.float32),
            pltpu.VMEM((3, B, YROWS, D_MODEL), jnp.float32),
            pltpu.VMEM((3, B, YROWS, D_MODEL), jnp.float32),
            pltpu.VMEM((1, B, XROWS, D_MODEL), jnp.float32),
            pltpu.VMEM((1, B, XROWS, D_MODEL), jnp.float32),
            pltpu.SemaphoreType.DMA((14,)),
            pltpu.SemaphoreType.DMA((14,)),
        ],
        compiler_params=pltpu.CompilerParams(collective_id=0),
    )(x, Wq, k_loc, v_loc, Wo)
